# baseline (speedup 1.0000x reference)
import functools

import jax
import jax.numpy as jnp
from jax import lax
from jax.experimental import pallas as pl
from jax.experimental.pallas import tpu as pltpu
from jax.experimental.pallas import tpu_sc as plsc

VOCAB = 1000000
EMBED_DIM = 64
BATCH = 16384
HIST = 200

_B = BATCH * HIST
_NC = 2
_NS = 16
_NW = _NC * _NS
_PER_W = _B // _NW
_CHUNK = 512
_NSTEP = _PER_W // _CHUNK


def _make_gather():
  mesh = plsc.VectorSubcoreMesh(core_axis_name="c", subcore_axis_name="s")

  @functools.partial(
      pl.kernel,
      mesh=mesh,
      out_type=jax.ShapeDtypeStruct((_B, EMBED_DIM), jnp.float32),
      compiler_params=pltpu.CompilerParams(use_tc_tiling_on_sc=False),
      scratch_types=[
          pltpu.VMEM((_CHUNK,), jnp.int32),
          pltpu.VMEM((_CHUNK, EMBED_DIM), jnp.float32),
          pltpu.SemaphoreType.DMA,
      ],
  )
  def gather_kernel(table_hbm, idx_hbm, out_hbm, idx_v, rows_v, sem):
    wid = lax.axis_index("s") * _NC + lax.axis_index("c")
    base = wid * _PER_W

    def step(g, carry):
      off = base + g * _CHUNK
      pltpu.sync_copy(idx_hbm.at[pl.ds(off, _CHUNK)], idx_v)
      pltpu.async_copy(table_hbm.at[idx_v], rows_v, sem).wait()
      pltpu.sync_copy(rows_v, out_hbm.at[pl.ds(off, _CHUNK)])
      return carry

    lax.fori_loop(0, _NSTEP, step, 0)

  return gather_kernel


_gather = _make_gather()


@jax.jit
def kernel(x, table):
  idx = x.astype(jnp.int32).reshape(-1)
  out = _gather(table, idx)
  return out.reshape(BATCH, HIST, EMBED_DIM)


# speedup vs baseline: 1.0743x; 1.0743x over previous
import functools

import jax
import jax.numpy as jnp
from jax import lax
from jax.experimental import pallas as pl
from jax.experimental.pallas import tpu as pltpu
from jax.experimental.pallas import tpu_sc as plsc

VOCAB = 1000000
EMBED_DIM = 64
BATCH = 16384
HIST = 200

_B = BATCH * HIST
_NC = 2
_NS = 16
_NW = _NC * _NS
_PER_W = _B // _NW
_C = 400
_NCHUNK = _PER_W // _C
_NB = 4
_NG = _NCHUNK // _NB


def _make_gather():
  mesh = plsc.VectorSubcoreMesh(core_axis_name="c", subcore_axis_name="s")

  @functools.partial(
      pl.kernel,
      mesh=mesh,
      out_type=jax.ShapeDtypeStruct((_B, EMBED_DIM), jnp.float32),
      compiler_params=pltpu.CompilerParams(use_tc_tiling_on_sc=False),
      scratch_types=[
          pltpu.VMEM((_NB, _C), jnp.int32),
          pltpu.VMEM((_NB, _C, EMBED_DIM), jnp.float32),
          pltpu.SemaphoreType.DMA((_NB,)),
          pltpu.SemaphoreType.DMA((_NB,)),
          pltpu.SemaphoreType.DMA((_NB,)),
      ],
  )
  def gather_kernel(table_hbm, idx_hbm, out_hbm, idx_v, rows_v, sem_i,
                    sem_g, sem_s):
    wid = lax.axis_index("s") * _NC + lax.axis_index("c")
    base = wid * _PER_W

    def idx_copy(b, c):
      return pltpu.make_async_copy(
          idx_hbm.at[pl.ds(base + c * _C, _C)], idx_v.at[b], sem_i.at[b])

    def gather(b):
      return pltpu.make_async_copy(
          table_hbm.at[idx_v.at[b]], rows_v.at[b], sem_g.at[b])

    def store(b, c):
      return pltpu.make_async_copy(
          rows_v.at[b], out_hbm.at[pl.ds(base + c * _C, _C)], sem_s.at[b])

    for b in range(_NB):
      idx_copy(b, b).start()
      idx_copy(b, b).wait()
      gather(b).start()
      if b >= 2:
        gather(b - 2).wait()
        store(b - 2, b - 2).start()
        idx_copy(b - 2, b + 2).start()

    def group(i, carry):
      for b in range(_NB):
        c = i * _NB + b
        store(b, c - _NB).wait()
        idx_copy(b, c).wait()
        gather(b).start()
        b2 = (b + 2) % _NB
        gather(b2).wait()
        store(b2, c - 2).start()

        @pl.when(c + 2 < _NCHUNK)
        def _():
          idx_copy(b2, c + 2).start()

      return carry

    lax.fori_loop(1, _NG, group, 0)

    last = (_NG - 1) * _NB
    for b in (2, 3):
      gather(b).wait()
      store(b, last + b).start()
    for b in range(_NB):
      store(b, last + b).wait()

  return gather_kernel


_gather = _make_gather()


@jax.jit
def kernel(x, table):
  idx = x.astype(jnp.int32).reshape(-1)
  out = _gather(table, idx)
  return out.reshape(BATCH, HIST, EMBED_DIM)


# speedup vs baseline: 1.7734x; 1.6508x over previous
import functools

import jax
import jax.numpy as jnp
from jax import lax
from jax.experimental import pallas as pl
from jax.experimental.pallas import tpu as pltpu
from jax.experimental.pallas import tpu_sc as plsc

VOCAB = 1000000
EMBED_DIM = 64
BATCH = 16384
HIST = 200
_PAD = 128

_NC = 2
_NS = 16
_NW = _NC * _NS
_PER_W = BATCH // _NW
_NB = 8
_LAG = 4
_NG = _PER_W // _NB


def _make_gather():
  mesh = plsc.VectorSubcoreMesh(core_axis_name="c", subcore_axis_name="s")

  @functools.partial(
      pl.kernel,
      mesh=mesh,
      out_type=jax.ShapeDtypeStruct((BATCH, HIST, _PAD), jnp.float32),
      compiler_params=pltpu.CompilerParams(use_tc_tiling_on_sc=False),
      scratch_types=[
          pltpu.VMEM((_NB, HIST), jnp.int32),
          pltpu.VMEM((_NB, HIST, EMBED_DIM), jnp.float32),
          pltpu.SemaphoreType.DMA((_NB,)),
          pltpu.SemaphoreType.DMA((_NB,)),
          pltpu.SemaphoreType.DMA((_NB,)),
      ],
  )
  def gather_kernel(table_hbm, x_hbm, out_hbm, idx_v, rows_v, sem_i,
                    sem_g, sem_s):
    wid = lax.axis_index("s") * _NC + lax.axis_index("c")
    base = wid * _PER_W

    def idx_copy(b, c):
      return pltpu.make_async_copy(x_hbm.at[base + c], idx_v.at[b], sem_i.at[b])

    def gather(b):
      return pltpu.make_async_copy(
          table_hbm.at[idx_v.at[b]], rows_v.at[b], sem_g.at[b])

    def store(b, c):
      return pltpu.make_async_copy(
          rows_v.at[b], out_hbm.at[base + c, :, pl.ds(0, EMBED_DIM)],
          sem_s.at[b])

    for b in range(_NB):
      idx_copy(b, b).start()
      idx_copy(b, b).wait()
      gather(b).start()
      if b >= _LAG:
        gather(b - _LAG).wait()
        store(b - _LAG, b - _LAG).start()
        idx_copy(b - _LAG, b - _LAG + _NB).start()

    def group(i, carry):
      for b in range(_NB):
        c = i * _NB + b
        store(b, c - _NB).wait()
        idx_copy(b, c).wait()
        gather(b).start()
        b2 = (b + _LAG) % _NB
        gather(b2).wait()
        store(b2, c - _LAG).start()

        @pl.when(c + _LAG < _PER_W)
        def _():
          idx_copy(b2, c + _LAG).start()

      return carry

    lax.fori_loop(1, _NG, group, 0)

    last = _PER_W - _NB
    for b in range(_NB - _LAG, _NB):
      gather(b).wait()
      store(b, last + b).start()
    for b in range(_NB):
      store(b, last + b).wait()

  return gather_kernel


_gather = _make_gather()


@jax.jit
def kernel(x, table):
  padded = _gather(table, x.astype(jnp.int32))
  return padded[:, :, :EMBED_DIM]
